# baseline (speedup 1.0000x reference)
import numpy as np
import jax
import jax.numpy as jnp
from jax import lax
from jax.experimental import pallas as pl

N_BINS = 15
ROWS = 50000
COLS = 1000
BLOCK = 1000
GRID = ROWS // BLOCK

_BOUNDS = np.linspace(0.0, 1.0, N_BINS + 1).astype(np.float32)
_LOWERS = np.concatenate([_BOUNDS[:-1], [2.0]]).astype(np.float32)
_UPPERS = np.concatenate([_BOUNDS[1:], [1.0]]).astype(np.float32)


def _ece_kernel(x_ref, lab_ref, lowers_ref, uppers_ref, cnt_ref, sconf_ref,
                sacc_ref, ece_ref):
    i = pl.program_id(0)

    x = x_ref[...]
    labv = lab_ref[0]

    m = jnp.max(x, axis=1, keepdims=True)
    s = jnp.sum(jnp.exp(x - m), axis=1, keepdims=True)
    conf = 1.0 / s

    col = lax.broadcasted_iota(jnp.int32, (BLOCK, COLS), 1)
    pred = jnp.min(jnp.where(x == m, col, COLS), axis=1, keepdims=True)
    acc = (pred == labv).astype(jnp.float32)

    lowers = lowers_ref[...]
    uppers = uppers_ref[...]
    mask = ((conf > lowers) & (conf <= uppers)).astype(jnp.float32)

    cnt = jnp.sum(mask, axis=0, keepdims=True)
    sconf = jnp.sum(mask * conf, axis=0, keepdims=True)
    sacc = jnp.sum(mask * acc, axis=0, keepdims=True)

    @pl.when(i == 0)
    def _init():
        cnt_ref[...] = cnt
        sconf_ref[...] = sconf
        sacc_ref[...] = sacc

    @pl.when(i != 0)
    def _accum():
        cnt_ref[...] += cnt
        sconf_ref[...] += sconf
        sacc_ref[...] += sacc

    @pl.when(i == GRID - 1)
    def _finalize():
        c = cnt_ref[...]
        safe = jnp.maximum(c, 1.0)
        avg_conf = sconf_ref[...] / safe
        avg_acc = sacc_ref[...] / safe
        prop = c / float(ROWS)
        per_bin = jnp.where(prop > 0.0, jnp.abs(avg_conf - avg_acc) * prop, 0.0)
        ece_ref[...] = jnp.sum(per_bin, keepdims=True).reshape(1, 1)


def kernel(logits, labels):
    labels3 = labels.astype(jnp.int32).reshape(GRID, BLOCK, 1)
    outs = pl.pallas_call(
        _ece_kernel,
        grid=(GRID,),
        in_specs=[
            pl.BlockSpec((BLOCK, COLS), lambda i: (i, 0)),
            pl.BlockSpec((1, BLOCK, 1), lambda i: (i, 0, 0)),
            pl.BlockSpec((1, 16), lambda i: (0, 0)),
            pl.BlockSpec((1, 16), lambda i: (0, 0)),
        ],
        out_specs=[
            pl.BlockSpec((1, 16), lambda i: (0, 0)),
            pl.BlockSpec((1, 16), lambda i: (0, 0)),
            pl.BlockSpec((1, 16), lambda i: (0, 0)),
            pl.BlockSpec((1, 1), lambda i: (0, 0)),
        ],
        out_shape=[
            jax.ShapeDtypeStruct((1, 16), jnp.float32),
            jax.ShapeDtypeStruct((1, 16), jnp.float32),
            jax.ShapeDtypeStruct((1, 16), jnp.float32),
            jax.ShapeDtypeStruct((1, 1), jnp.float32),
        ],
    )(logits, labels3, jnp.asarray(_LOWERS).reshape(1, 16),
      jnp.asarray(_UPPERS).reshape(1, 16))
    return outs[3].reshape(1)


# speedup vs baseline: 1.0392x; 1.0392x over previous
import numpy as np
import jax
import jax.numpy as jnp
from jax import lax
from jax.experimental import pallas as pl

N_BINS = 15
ROWS = 50000
COLS = 1000
BLOCK = 2000
GRID = ROWS // BLOCK

_BOUNDS = np.linspace(0.0, 1.0, N_BINS + 1).astype(np.float32)
_LOWERS = np.concatenate([_BOUNDS[:-1], [2.0]]).astype(np.float32)
_UPPERS = np.concatenate([_BOUNDS[1:], [1.0]]).astype(np.float32)


def _ece_kernel(x_ref, lab_ref, lowers_ref, uppers_ref, cnt_ref, sconf_ref,
                sacc_ref, ece_ref):
    i = pl.program_id(0)

    x = x_ref[...]
    labv = lab_ref[0]

    m = jnp.max(x, axis=1, keepdims=True)
    s = jnp.sum(jnp.exp(x - m), axis=1, keepdims=True)
    conf = 1.0 / s

    col = lax.broadcasted_iota(jnp.int32, (BLOCK, COLS), 1)
    pred = jnp.min(jnp.where(x == m, col, COLS), axis=1, keepdims=True)
    acc = (pred == labv).astype(jnp.float32)

    lowers = lowers_ref[...]
    uppers = uppers_ref[...]
    mask = ((conf > lowers) & (conf <= uppers)).astype(jnp.float32)

    cnt = jnp.sum(mask, axis=0, keepdims=True)
    sconf = jnp.sum(mask * conf, axis=0, keepdims=True)
    sacc = jnp.sum(mask * acc, axis=0, keepdims=True)

    @pl.when(i == 0)
    def _init():
        cnt_ref[...] = cnt
        sconf_ref[...] = sconf
        sacc_ref[...] = sacc

    @pl.when(i != 0)
    def _accum():
        cnt_ref[...] += cnt
        sconf_ref[...] += sconf
        sacc_ref[...] += sacc

    @pl.when(i == GRID - 1)
    def _finalize():
        c = cnt_ref[...]
        safe = jnp.maximum(c, 1.0)
        avg_conf = sconf_ref[...] / safe
        avg_acc = sacc_ref[...] / safe
        prop = c / float(ROWS)
        per_bin = jnp.where(prop > 0.0, jnp.abs(avg_conf - avg_acc) * prop, 0.0)
        ece_ref[...] = jnp.sum(per_bin, keepdims=True).reshape(1, 1)


def kernel(logits, labels):
    labels3 = labels.astype(jnp.int32).reshape(GRID, BLOCK, 1)
    outs = pl.pallas_call(
        _ece_kernel,
        grid=(GRID,),
        in_specs=[
            pl.BlockSpec((BLOCK, COLS), lambda i: (i, 0)),
            pl.BlockSpec((1, BLOCK, 1), lambda i: (i, 0, 0)),
            pl.BlockSpec((1, 16), lambda i: (0, 0)),
            pl.BlockSpec((1, 16), lambda i: (0, 0)),
        ],
        out_specs=[
            pl.BlockSpec((1, 16), lambda i: (0, 0)),
            pl.BlockSpec((1, 16), lambda i: (0, 0)),
            pl.BlockSpec((1, 16), lambda i: (0, 0)),
            pl.BlockSpec((1, 1), lambda i: (0, 0)),
        ],
        out_shape=[
            jax.ShapeDtypeStruct((1, 16), jnp.float32),
            jax.ShapeDtypeStruct((1, 16), jnp.float32),
            jax.ShapeDtypeStruct((1, 16), jnp.float32),
            jax.ShapeDtypeStruct((1, 1), jnp.float32),
        ],
    )(logits, labels3, jnp.asarray(_LOWERS).reshape(1, 16),
      jnp.asarray(_UPPERS).reshape(1, 16))
    return outs[3].reshape(1)
